# baseline (speedup 1.0000x reference)
import jax
import jax.numpy as jnp
from jax import lax
from jax.experimental import pallas as pl
from jax.experimental.pallas import tpu as pltpu

P = 128
K = 7
BB = 32


def _mm(a, b):
    return lax.dot_general(a, b, (((1,), (0,)), ((), ())),
                           preferred_element_type=jnp.float32)


def _topk_sel(pd):
    R = pd.shape[0]
    col = lax.broadcasted_iota(jnp.int32, (R, P), 1).astype(jnp.float32)
    sels = []
    for t in range(K + 1):
        mx = jnp.max(pd, axis=1, keepdims=True)
        cand = jnp.where(pd == mx, col, 1e9)
        idx = jnp.min(cand, axis=1, keepdims=True)
        if t == 0:
            pd = jnp.where(col == idx, -1e30, pd)
        else:
            sel = (col == idx).astype(jnp.float32)
            sels.append(sel)
            if t < K:
                pd = pd - sel * 1e30
    return sels


def _edge_conv(pts, fts, w0a_m_b, w0b, b0, layers, shortcut):
    C = fts.shape[0]
    pts2 = pts + pts
    xx = jnp.sum(pts * pts, axis=0, keepdims=True)
    pds = []
    for b in range(BB):
        inner2 = lax.dot_general(pts2[:, b * P:(b + 1) * P],
                                 pts[:, b * P:(b + 1) * P],
                                 (((0,), (0,)), ((), ())),
                                 preferred_element_type=jnp.float32)
        pds.append(inner2 - xx[:, b * P:(b + 1) * P])
    pd = jnp.concatenate(pds, axis=0)

    sels = _topk_sel(pd)
    u = _mm(w0b, fts)
    v = _mm(w0a_m_b, fts) + b0
    y_parts = []
    for b in range(BB):
        ub = u[:, b * P:(b + 1) * P]
        vb = v[:, b * P:(b + 1) * P]
        for s in sels:
            g = lax.dot_general(ub, s[b * P:(b + 1) * P],
                                (((1,), (1,)), ((), ())),
                                preferred_element_type=jnp.float32)
            y_parts.append(jnp.maximum(g + vb, 0.0))
    y = jnp.concatenate(y_parts, axis=1)
    for w, bb_ in layers:
        y = jnp.maximum(_mm(w, y) + bb_, 0.0)

    O = y.shape[0]
    mean_parts = []
    for b in range(BB):
        blk = y[:, b * K * P:(b + 1) * K * P]
        acc = blk[:, 0:P]
        for t in range(1, K):
            acc = acc + blk[:, t * P:(t + 1) * P]
        mean_parts.append(acc)
    mean = jnp.concatenate(mean_parts, axis=1) * (1.0 / K)

    if shortcut is None:
        scv = fts
    else:
        wsc, bsc = shortcut
        scv = _mm(wsc, fts) + bsc
    return jnp.maximum(scv + mean, 0.0)


def _net_body(pts_ref, f_ref, m_ref,
              a0_ref, c0_ref, sfts_ref, bfts_ref,
              e1w0a_ref, e1w0b_ref, e1b0_ref, e1w1_ref, e1b1_ref,
              e1w2_ref, e1b2_ref,
              e2w0a_ref, e2w0b_ref, e2b0_ref, e2w1_ref, e2b1_ref,
              e2w2_ref, e2b2_ref,
              scw_ref, scb_ref, fusw_ref, fusb_ref,
              fc1w_ref, fc1b_ref, fc2w_ref, fc2b_ref,
              out_ref):
    f = f_ref[...]
    m = m_ref[...]
    pts_in = pts_ref[...]

    x = f * m
    fts0 = jnp.maximum(_mm(a0_ref[...], x) + c0_ref[...], 0.0)
    features = fts0 * m
    points = pts_in * m
    coord_shift = jnp.where(m == 0.0, 1e9, 0.0)
    fts = (features * sfts_ref[...] + bfts_ref[...]) * m

    out1 = _edge_conv(
        points + coord_shift, fts,
        e1w0a_ref[...], e1w0b_ref[...], e1b0_ref[...],
        [(e1w1_ref[...], e1b1_ref[...]), (e1w2_ref[...], e1b2_ref[...])],
        None) * m
    out2 = _edge_conv(
        out1 + coord_shift, out1,
        e2w0a_ref[...], e2w0b_ref[...], e2b0_ref[...],
        [(e2w1_ref[...], e2b1_ref[...]), (e2w2_ref[...], e2b2_ref[...])],
        (scw_ref[...], scb_ref[...])) * m

    cat = jnp.concatenate([out1, out2], axis=0)
    ff = jnp.maximum(_mm(fusw_ref[...], cat) + fusb_ref[...], 0.0) * m

    pooled_parts = []
    cnt_parts = []
    for b in range(BB):
        pooled_parts.append(
            jnp.sum(ff[:, b * P:(b + 1) * P], axis=1, keepdims=True))
        cnt_parts.append(
            jnp.sum(m[:, b * P:(b + 1) * P], axis=1, keepdims=True))
    pooled = jnp.concatenate(pooled_parts, axis=1)
    counts = jnp.maximum(jnp.concatenate(cnt_parts, axis=1), 1.0)
    pooled = pooled / counts

    h = jnp.maximum(_mm(fc1w_ref[...], pooled) + fc1b_ref[...], 0.0)
    o = _mm(fc2w_ref[...], h) + fc2b_ref[...]
    out_ref[0] = o


@jax.jit
def kernel(pf_points, pf_features, pf_mask, params):
    p = params
    B = pf_points.shape[0]
    inv = 1.0 / jnp.sqrt(1.0 + 1e-5)

    def fold(g, w):
        return (g * inv)[:, None] * w

    s0 = p['fcv_bn0_g'] * inv
    s1 = p['fcv_bn1_g'] * inv
    a0 = (s1[:, None] * p['fcv_w']) * s0[None, :]
    c0 = (s1 * (p['fcv_w'] @ p['fcv_bn0_b']) + p['fcv_bn1_b'])[:, None]
    sfts = (p['bn_fts_g'] * inv)[:, None]
    bfts = p['bn_fts_b'][:, None]

    def split_w0(g, w0):
        wf = fold(g, w0)
        c = wf.shape[1] // 2
        w0a, w0b = wf[:, :c], wf[:, c:]
        return w0a - w0b, w0b

    e1w0a, e1w0b = split_w0(p['ec1_g0'], p['ec1_w0'])
    e2w0a, e2w0b = split_w0(p['ec2_g0'], p['ec2_w0'])
    e1w1 = fold(p['ec1_g1'], p['ec1_w1'])
    e1w2 = fold(p['ec1_g2'], p['ec1_w2'])
    e2w1 = fold(p['ec2_g1'], p['ec2_w1'])
    e2w2 = fold(p['ec2_g2'], p['ec2_w2'])
    scw = fold(p['ec2_scg'], p['ec2_scw'])
    fusw = fold(p['fus_g'], p['fus_w'])
    b = lambda v: v[:, None]

    ops = [
        a0, c0, sfts, bfts,
        e1w0a, e1w0b, b(p['ec1_b0']), e1w1, b(p['ec1_b1']),
        e1w2, b(p['ec1_b2']),
        e2w0a, e2w0b, b(p['ec2_b0']), e2w1, b(p['ec2_b1']),
        e2w2, b(p['ec2_b2']),
        scw, b(p['ec2_scb']), fusw, b(p['fus_b']),
        p['fc1_w'], b(p['fc1_b']), p['fc2_w'], b(p['fc2_b']),
    ]

    pts_f = pf_points.transpose(1, 0, 2).reshape(2, B * P)
    f_f = pf_features.transpose(1, 0, 2).reshape(5, B * P)
    m_f = pf_mask.transpose(1, 0, 2).reshape(1, B * P)

    grid = (B // BB,)
    col_spec = lambda c: pl.BlockSpec((c, BB * P), lambda i: (0, i))
    full = lambda arr: pl.BlockSpec(arr.shape, lambda i: (0,) * arr.ndim)

    out = pl.pallas_call(
        _net_body,
        grid=grid,
        in_specs=[col_spec(2), col_spec(5), col_spec(1)]
        + [full(o) for o in ops],
        out_specs=pl.BlockSpec((1, 2, BB), lambda i: (i, 0, 0)),
        out_shape=jax.ShapeDtypeStruct((B // BB, 2, BB), jnp.float32),
        compiler_params=pltpu.CompilerParams(
            dimension_semantics=("arbitrary",)),
    )(pts_f, f_f, m_f, *ops)
    return out.transpose(0, 2, 1).reshape(B, 2)


# speedup vs baseline: 1.0668x; 1.0668x over previous
import jax
import jax.numpy as jnp
from jax import lax
from jax.experimental import pallas as pl
from jax.experimental.pallas import tpu as pltpu

P = 128
K = 7
BB = 32


def _mm(a, b):
    return lax.dot_general(a, b, (((1,), (0,)), ((), ())),
                           preferred_element_type=jnp.float32)


def _topk_sel(pd):
    R = pd.shape[0]
    col = lax.broadcasted_iota(jnp.int32, (R, P), 1).astype(jnp.float32)
    sels = []
    for t in range(K + 1):
        mx = jnp.max(pd, axis=1, keepdims=True)
        cand = jnp.where(pd == mx, col, 1e9)
        idx = jnp.min(cand, axis=1, keepdims=True)
        sel = (col == idx).astype(jnp.float32)
        if t > 0:
            sels.append(sel)
        if t < K:
            pd = pd - sel * 1e30
    return sels


def _edge_conv(pts, fts, w0a_m_b, w0b, b0, layers, shortcut):
    C = fts.shape[0]
    xx = jnp.sum(pts * pts, axis=0, keepdims=True)
    pds = []
    for b in range(BB):
        pb = pts[:, b * P:(b + 1) * P]
        inner = lax.dot_general(pb, pb, (((0,), (0,)), ((), ())),
                                preferred_element_type=jnp.float32)
        pds.append(2.0 * inner - xx[:, b * P:(b + 1) * P])
    pd = jnp.concatenate(pds, axis=0)

    sels = _topk_sel(pd)
    u = _mm(w0b, fts)
    v = _mm(w0a_m_b, fts) + b0
    y_parts = []
    for b in range(BB):
        ub = u[:, b * P:(b + 1) * P]
        vb = v[:, b * P:(b + 1) * P]
        for s in sels:
            g = lax.dot_general(ub, s[b * P:(b + 1) * P],
                                (((1,), (1,)), ((), ())),
                                preferred_element_type=jnp.float32)
            y_parts.append(jnp.maximum(g + vb, 0.0))
    y = jnp.concatenate(y_parts, axis=1)
    for w, bb_ in layers:
        y = jnp.maximum(_mm(w, y) + bb_, 0.0)

    O = y.shape[0]
    mean_parts = []
    for b in range(BB):
        blk = y[:, b * K * P:(b + 1) * K * P]
        acc = blk[:, 0:P]
        for t in range(1, K):
            acc = acc + blk[:, t * P:(t + 1) * P]
        mean_parts.append(acc)
    mean = jnp.concatenate(mean_parts, axis=1) * (1.0 / K)

    if shortcut is None:
        scv = fts
    else:
        wsc, bsc = shortcut
        scv = _mm(wsc, fts) + bsc
    return jnp.maximum(scv + mean, 0.0)


def _net_body(pts_ref, f_ref, m_ref,
              a0_ref, c0_ref, sfts_ref, bfts_ref,
              e1w0a_ref, e1w0b_ref, e1b0_ref, e1w1_ref, e1b1_ref,
              e1w2_ref, e1b2_ref,
              e2w0a_ref, e2w0b_ref, e2b0_ref, e2w1_ref, e2b1_ref,
              e2w2_ref, e2b2_ref,
              scw_ref, scb_ref, fusw_ref, fusb_ref,
              fc1w_ref, fc1b_ref, fc2w_ref, fc2b_ref,
              out_ref):
    f = f_ref[...]
    m = m_ref[...]
    pts_in = pts_ref[...]

    x = f * m
    fts0 = jnp.maximum(_mm(a0_ref[...], x) + c0_ref[...], 0.0)
    features = fts0 * m
    points = pts_in * m
    coord_shift = jnp.where(m == 0.0, 1e9, 0.0)
    fts = (features * sfts_ref[...] + bfts_ref[...]) * m

    out1 = _edge_conv(
        points + coord_shift, fts,
        e1w0a_ref[...], e1w0b_ref[...], e1b0_ref[...],
        [(e1w1_ref[...], e1b1_ref[...]), (e1w2_ref[...], e1b2_ref[...])],
        None) * m
    out2 = _edge_conv(
        out1 + coord_shift, out1,
        e2w0a_ref[...], e2w0b_ref[...], e2b0_ref[...],
        [(e2w1_ref[...], e2b1_ref[...]), (e2w2_ref[...], e2b2_ref[...])],
        (scw_ref[...], scb_ref[...])) * m

    cat = jnp.concatenate([out1, out2], axis=0)
    ff = jnp.maximum(_mm(fusw_ref[...], cat) + fusb_ref[...], 0.0) * m

    pooled_parts = []
    cnt_parts = []
    for b in range(BB):
        pooled_parts.append(
            jnp.sum(ff[:, b * P:(b + 1) * P], axis=1, keepdims=True))
        cnt_parts.append(
            jnp.sum(m[:, b * P:(b + 1) * P], axis=1, keepdims=True))
    pooled = jnp.concatenate(pooled_parts, axis=1)
    counts = jnp.maximum(jnp.concatenate(cnt_parts, axis=1), 1.0)
    pooled = pooled / counts

    h = jnp.maximum(_mm(fc1w_ref[...], pooled) + fc1b_ref[...], 0.0)
    o = _mm(fc2w_ref[...], h) + fc2b_ref[...]
    out_ref[0] = o


@jax.jit
def kernel(pf_points, pf_features, pf_mask, params):
    p = params
    B = pf_points.shape[0]
    inv = 1.0 / jnp.sqrt(1.0 + 1e-5)

    def fold(g, w):
        return (g * inv)[:, None] * w

    s0 = p['fcv_bn0_g'] * inv
    s1 = p['fcv_bn1_g'] * inv
    a0 = (s1[:, None] * p['fcv_w']) * s0[None, :]
    c0 = (s1 * (p['fcv_w'] @ p['fcv_bn0_b']) + p['fcv_bn1_b'])[:, None]
    sfts = (p['bn_fts_g'] * inv)[:, None]
    bfts = p['bn_fts_b'][:, None]

    def split_w0(g, w0):
        wf = fold(g, w0)
        c = wf.shape[1] // 2
        w0a, w0b = wf[:, :c], wf[:, c:]
        return w0a - w0b, w0b

    e1w0a, e1w0b = split_w0(p['ec1_g0'], p['ec1_w0'])
    e2w0a, e2w0b = split_w0(p['ec2_g0'], p['ec2_w0'])
    e1w1 = fold(p['ec1_g1'], p['ec1_w1'])
    e1w2 = fold(p['ec1_g2'], p['ec1_w2'])
    e2w1 = fold(p['ec2_g1'], p['ec2_w1'])
    e2w2 = fold(p['ec2_g2'], p['ec2_w2'])
    scw = fold(p['ec2_scg'], p['ec2_scw'])
    fusw = fold(p['fus_g'], p['fus_w'])
    b = lambda v: v[:, None]

    ops = [
        a0, c0, sfts, bfts,
        e1w0a, e1w0b, b(p['ec1_b0']), e1w1, b(p['ec1_b1']),
        e1w2, b(p['ec1_b2']),
        e2w0a, e2w0b, b(p['ec2_b0']), e2w1, b(p['ec2_b1']),
        e2w2, b(p['ec2_b2']),
        scw, b(p['ec2_scb']), fusw, b(p['fus_b']),
        p['fc1_w'], b(p['fc1_b']), p['fc2_w'], b(p['fc2_b']),
    ]

    pts_f = pf_points.transpose(1, 0, 2).reshape(2, B * P)
    f_f = pf_features.transpose(1, 0, 2).reshape(5, B * P)
    m_f = pf_mask.transpose(1, 0, 2).reshape(1, B * P)

    grid = (B // BB,)
    col_spec = lambda c: pl.BlockSpec((c, BB * P), lambda i: (0, i))
    full = lambda arr: pl.BlockSpec(arr.shape, lambda i: (0,) * arr.ndim)

    out = pl.pallas_call(
        _net_body,
        grid=grid,
        in_specs=[col_spec(2), col_spec(5), col_spec(1)]
        + [full(o) for o in ops],
        out_specs=pl.BlockSpec((1, 2, BB), lambda i: (i, 0, 0)),
        out_shape=jax.ShapeDtypeStruct((B // BB, 2, BB), jnp.float32),
        compiler_params=pltpu.CompilerParams(
            dimension_semantics=("arbitrary",)),
    )(pts_f, f_f, m_f, *ops)
    return out.transpose(0, 2, 1).reshape(B, 2)


# speedup vs baseline: 1.2046x; 1.1292x over previous
import jax
import jax.numpy as jnp
from jax import lax
from jax.experimental import pallas as pl
from jax.experimental.pallas import tpu as pltpu

P = 128
K = 7
BB = 32


def _mm(a, b):
    return lax.dot_general(a, b, (((1,), (0,)), ((), ())),
                           preferred_element_type=jnp.float32)


def _edge_conv(pts, fts, w0a_m_b, w0b, b0, layers, shortcut):
    xx = jnp.sum(pts * pts, axis=0, keepdims=True)
    pds = []
    for b in range(BB):
        pb = pts[:, b * P:(b + 1) * P]
        inner = lax.dot_general(pb, pb, (((0,), (0,)), ((), ())),
                                preferred_element_type=jnp.float32)
        pds.append(2.0 * inner - xx[:, b * P:(b + 1) * P])
    pd = jnp.concatenate(pds, axis=0)

    u = _mm(w0b, fts)
    v = _mm(w0a_m_b, fts) + b0

    R = pd.shape[0]
    col = lax.broadcasted_iota(jnp.int32, (R, P), 1).astype(jnp.float32)
    y_parts = []
    for t in range(K + 1):
        mx = jnp.max(pd, axis=1, keepdims=True)
        cand = jnp.where(pd == mx, col, 1e9)
        idx = jnp.min(cand, axis=1, keepdims=True)
        sel = (col == idx).astype(jnp.float32)
        if t > 0:
            for b in range(BB):
                g = lax.dot_general(u[:, b * P:(b + 1) * P],
                                    sel[b * P:(b + 1) * P],
                                    (((1,), (1,)), ((), ())),
                                    preferred_element_type=jnp.float32)
                y_parts.append(
                    jnp.maximum(g + v[:, b * P:(b + 1) * P], 0.0))
        if t < K:
            pd = pd - sel * 1e30

    y = jnp.concatenate(y_parts, axis=1)
    for w, bb_ in layers:
        y = jnp.maximum(_mm(w, y) + bb_, 0.0)

    acc = y[:, 0:BB * P]
    for t in range(1, K):
        acc = acc + y[:, t * BB * P:(t + 1) * BB * P]
    mean = acc * (1.0 / K)

    if shortcut is None:
        scv = fts
    else:
        wsc, bsc = shortcut
        scv = _mm(wsc, fts) + bsc
    return jnp.maximum(scv + mean, 0.0)


def _net_body(pts_ref, f_ref, m_ref,
              a0_ref, c0_ref, sfts_ref, bfts_ref,
              e1w0a_ref, e1w0b_ref, e1b0_ref, e1w1_ref, e1b1_ref,
              e1w2_ref, e1b2_ref,
              e2w0a_ref, e2w0b_ref, e2b0_ref, e2w1_ref, e2b1_ref,
              e2w2_ref, e2b2_ref,
              scw_ref, scb_ref, fusw_ref, fusb_ref,
              fc1w_ref, fc1b_ref, fc2w_ref, fc2b_ref,
              out_ref):
    f = f_ref[...]
    m = m_ref[...]
    pts_in = pts_ref[...]

    x = f * m
    fts0 = jnp.maximum(_mm(a0_ref[...], x) + c0_ref[...], 0.0)
    features = fts0 * m
    points = pts_in * m
    coord_shift = jnp.where(m == 0.0, 1e9, 0.0)
    fts = (features * sfts_ref[...] + bfts_ref[...]) * m

    out1 = _edge_conv(
        points + coord_shift, fts,
        e1w0a_ref[...], e1w0b_ref[...], e1b0_ref[...],
        [(e1w1_ref[...], e1b1_ref[...]), (e1w2_ref[...], e1b2_ref[...])],
        None) * m
    out2 = _edge_conv(
        out1 + coord_shift, out1,
        e2w0a_ref[...], e2w0b_ref[...], e2b0_ref[...],
        [(e2w1_ref[...], e2b1_ref[...]), (e2w2_ref[...], e2b2_ref[...])],
        (scw_ref[...], scb_ref[...])) * m

    cat = jnp.concatenate([out1, out2], axis=0)
    ff = jnp.maximum(_mm(fusw_ref[...], cat) + fusb_ref[...], 0.0) * m

    pooled_parts = []
    cnt_parts = []
    for b in range(BB):
        pooled_parts.append(
            jnp.sum(ff[:, b * P:(b + 1) * P], axis=1, keepdims=True))
        cnt_parts.append(
            jnp.sum(m[:, b * P:(b + 1) * P], axis=1, keepdims=True))
    pooled = jnp.concatenate(pooled_parts, axis=1)
    counts = jnp.maximum(jnp.concatenate(cnt_parts, axis=1), 1.0)
    pooled = pooled / counts

    h = jnp.maximum(_mm(fc1w_ref[...], pooled) + fc1b_ref[...], 0.0)
    o = _mm(fc2w_ref[...], h) + fc2b_ref[...]
    out_ref[0] = o


@jax.jit
def kernel(pf_points, pf_features, pf_mask, params):
    p = params
    B = pf_points.shape[0]
    inv = 1.0 / jnp.sqrt(1.0 + 1e-5)

    def fold(g, w):
        return (g * inv)[:, None] * w

    s0 = p['fcv_bn0_g'] * inv
    s1 = p['fcv_bn1_g'] * inv
    a0 = (s1[:, None] * p['fcv_w']) * s0[None, :]
    c0 = (s1 * (p['fcv_w'] @ p['fcv_bn0_b']) + p['fcv_bn1_b'])[:, None]
    sfts = (p['bn_fts_g'] * inv)[:, None]
    bfts = p['bn_fts_b'][:, None]

    def split_w0(g, w0):
        wf = fold(g, w0)
        c = wf.shape[1] // 2
        w0a, w0b = wf[:, :c], wf[:, c:]
        return w0a - w0b, w0b

    e1w0a, e1w0b = split_w0(p['ec1_g0'], p['ec1_w0'])
    e2w0a, e2w0b = split_w0(p['ec2_g0'], p['ec2_w0'])
    e1w1 = fold(p['ec1_g1'], p['ec1_w1'])
    e1w2 = fold(p['ec1_g2'], p['ec1_w2'])
    e2w1 = fold(p['ec2_g1'], p['ec2_w1'])
    e2w2 = fold(p['ec2_g2'], p['ec2_w2'])
    scw = fold(p['ec2_scg'], p['ec2_scw'])
    fusw = fold(p['fus_g'], p['fus_w'])
    b = lambda v: v[:, None]

    ops = [
        a0, c0, sfts, bfts,
        e1w0a, e1w0b, b(p['ec1_b0']), e1w1, b(p['ec1_b1']),
        e1w2, b(p['ec1_b2']),
        e2w0a, e2w0b, b(p['ec2_b0']), e2w1, b(p['ec2_b1']),
        e2w2, b(p['ec2_b2']),
        scw, b(p['ec2_scb']), fusw, b(p['fus_b']),
        p['fc1_w'], b(p['fc1_b']), p['fc2_w'], b(p['fc2_b']),
    ]

    pts_f = pf_points.transpose(1, 0, 2).reshape(2, B * P)
    f_f = pf_features.transpose(1, 0, 2).reshape(5, B * P)
    m_f = pf_mask.transpose(1, 0, 2).reshape(1, B * P)

    grid = (B // BB,)
    col_spec = lambda c: pl.BlockSpec((c, BB * P), lambda i: (0, i))
    full = lambda arr: pl.BlockSpec(arr.shape, lambda i: (0,) * arr.ndim)

    out = pl.pallas_call(
        _net_body,
        grid=grid,
        in_specs=[col_spec(2), col_spec(5), col_spec(1)]
        + [full(o) for o in ops],
        out_specs=pl.BlockSpec((1, 2, BB), lambda i: (i, 0, 0)),
        out_shape=jax.ShapeDtypeStruct((B // BB, 2, BB), jnp.float32),
        compiler_params=pltpu.CompilerParams(
            dimension_semantics=("arbitrary",)),
    )(pts_f, f_f, m_f, *ops)
    return out.transpose(0, 2, 1).reshape(B, 2)
